# baseline (speedup 1.0000x reference)
import functools

import jax
import jax.numpy as jnp
from jax import lax
from jax.experimental import pallas as pl
from jax.experimental.pallas import tpu as pltpu
from jax.experimental.pallas import tpu_sc as plsc

_N, _E, _D, _DM, _DOUT = 10000, 640000, 128, 16, 16


def _node_proj_body(h_ref, v_ref, wh_ref, wvs_ref, wvd_ref, b_ref, a_ref, bt_ref):
    hW = jnp.dot(h_ref[...], wh_ref[...], preferred_element_type=jnp.float32)
    a_ref[...] = hW + jnp.dot(v_ref[...], wvs_ref[...],
                              preferred_element_type=jnp.float32)
    bt_ref[...] = (hW + jnp.dot(v_ref[...], wvd_ref[...],
                                preferred_element_type=jnp.float32)
                   + b_ref[...])


_NODE_BLK = 2000

_node_proj = pl.pallas_call(
    _node_proj_body,
    grid=(_N // _NODE_BLK,),
    in_specs=[
        pl.BlockSpec((_NODE_BLK, _D), lambda i: (i, 0)),
        pl.BlockSpec((_NODE_BLK, _D), lambda i: (i, 0)),
        pl.BlockSpec((_D, _DOUT), lambda i: (0, 0)),
        pl.BlockSpec((_D, _DOUT), lambda i: (0, 0)),
        pl.BlockSpec((_D, _DOUT), lambda i: (0, 0)),
        pl.BlockSpec((1, _DOUT), lambda i: (0, 0)),
    ],
    out_specs=[
        pl.BlockSpec((_NODE_BLK, _DOUT), lambda i: (i, 0)),
        pl.BlockSpec((_NODE_BLK, _DOUT), lambda i: (i, 0)),
    ],
    out_shape=[
        jax.ShapeDtypeStruct((_N, _DOUT), jnp.float32),
        jax.ShapeDtypeStruct((_N, _DOUT), jnp.float32),
    ],
)


_NW = 32
_CHUNK = 1024
_P = _CHUNK // 8
_NCHUNKS = _E // _CHUNK
_CPW = -(-_NCHUNKS // _NW)

_sc_mesh = plsc.VectorSubcoreMesh(core_axis_name="c", subcore_axis_name="s")


def _make_sc(chunk_off, nchunks):
    nfull = nchunks // _NW
    extra = nchunks - nfull * _NW

    @functools.partial(
        pl.kernel,
        out_type=jax.ShapeDtypeStruct((nchunks * _P, 128), jnp.float32),
        mesh=_sc_mesh,
        scratch_types=[
            pltpu.VMEM((_CHUNK,), jnp.int32),
            pltpu.VMEM((_CHUNK,), jnp.int32),
            pltpu.VMEM((_CHUNK,), jnp.int32),
            pltpu.VMEM((_CHUNK,), jnp.int32),
            pltpu.VMEM((_CHUNK, _DOUT), jnp.float32),
            pltpu.VMEM((_CHUNK, _DOUT), jnp.float32),
            pltpu.VMEM((_CHUNK, _DOUT), jnp.float32),
            pltpu.VMEM((_CHUNK, _DOUT), jnp.float32),
            pltpu.VMEM((_P, 128), jnp.float32),
            pltpu.VMEM((_P, 128), jnp.float32),
            pltpu.SemaphoreType.DMA,
            pltpu.SemaphoreType.DMA,
            pltpu.SemaphoreType.DMA,
            pltpu.SemaphoreType.DMA,
            pltpu.SemaphoreType.DMA,
            pltpu.SemaphoreType.DMA,
        ],
        compiler_params=pltpu.CompilerParams(use_tc_tiling_on_sc=False, needs_layout_passes=False),
    )
    def _sc_gather_sum(a_hbm, b_hbm, et_hbm, out_hbm,
                       idxs0, idxd0, idxs1, idxd1, bufa0, bufb0, bufa1, bufb1,
                       bufo0, bufo1, sem_i0, sem_i1, sem_g0, sem_g1, sem_o0, sem_o1):
        wid = lax.axis_index("s") * 2 + lax.axis_index("c")
        nw = jnp.where(wid < extra, nfull + 1, nfull)
        start = wid * nfull + jnp.minimum(wid, extra)
        idxs = (idxs0, idxs1)
        idxd = (idxd0, idxd1)
        bufa = (bufa0, bufa1)
        bufb = (bufb0, bufb1)
        bufo = (bufo0, bufo1)
        sem_i = (sem_i0, sem_i1)
        sem_g = (sem_g0, sem_g1)
        sem_o = (sem_o0, sem_o1)

        def issue_idx(p, ci):
            base = (chunk_off + start + ci) * _CHUNK
            pltpu.async_copy(et_hbm.at[0, pl.ds(base, _CHUNK)], idxs[p], sem_i[p])
            pltpu.async_copy(et_hbm.at[1, pl.ds(base, _CHUNK)], idxd[p], sem_i[p])

        def wait_idx(p):
            pltpu.make_async_copy(et_hbm.at[0, pl.ds(0, _CHUNK)], idxs[p], sem_i[p]).wait()
            pltpu.make_async_copy(et_hbm.at[1, pl.ds(0, _CHUNK)], idxd[p], sem_i[p]).wait()

        def issue_gather(p):
            pltpu.async_copy(a_hbm.at[idxs[p]], bufa[p], sem_g[p])
            pltpu.async_copy(b_hbm.at[idxd[p]], bufb[p], sem_g[p])

        def wait_gather(p):
            pltpu.make_async_copy(a_hbm.at[pl.ds(0, _CHUNK)], bufa[p], sem_g[p]).wait()
            pltpu.make_async_copy(b_hbm.at[pl.ds(0, _CHUNK)], bufb[p], sem_g[p]).wait()

        def issue_out(p, ci):
            pltpu.async_copy(bufo[p], out_hbm.at[pl.ds((start + ci) * _P, _P)], sem_o[p])

        def wait_out(p):
            pltpu.make_async_copy(bufo[p], out_hbm.at[pl.ds(0, _P)], sem_o[p]).wait()

        issue_idx(0, 0)
        wait_idx(0)
        issue_gather(0)
        issue_idx(1, 1)

        def pair_body(ci2, carry):
            for p in (0, 1):
                ci = ci2 * 2 + p

                @pl.when(ci < nw)
                def _():
                    op = 1 - p

                    @pl.when(ci + 1 < nw)
                    def _():
                        wait_idx(op)
                        issue_gather(op)

                    wait_gather(p)

                    @pl.when(ci + 2 < nw)
                    def _():
                        issue_idx(p, ci + 2)

                    def row_body(q, c2):
                        for k in range(8):
                            bufo[p][q, pl.ds(k * _DOUT, _DOUT)] = (
                                bufa[p][k * _P + q, :] + bufb[p][k * _P + q, :])
                        return c2

                    lax.fori_loop(0, _P, row_body, 0)

                    @pl.when(ci >= 2)
                    def _():
                        wait_out(p)

                    issue_out(p, ci)

            return carry

        lax.fori_loop(0, (nfull + 2) // 2, pair_body, 0)
        wait_out(0)
        wait_out(1)

    return _sc_gather_sum


_PART0_CHUNKS, _PART1_CHUNKS = 325, 300
_sc_part0 = _make_sc(0, _PART0_CHUNKS)
_sc_part1 = _make_sc(_PART0_CHUNKS, _PART1_CHUNKS)


_FMT_BLK = 12800


def _edge_fmt_body(ed_ref, t_ref):
    t_ref[...] = ed_ref[...].T


_edge_fmt = pl.pallas_call(
    _edge_fmt_body,
    grid=(_E // _FMT_BLK,),
    in_specs=[pl.BlockSpec((_FMT_BLK, 2), lambda i: (i, 0))],
    out_specs=pl.BlockSpec((2, _FMT_BLK), lambda i: (0, i)),
    out_shape=jax.ShapeDtypeStruct((2, _E), jnp.int32),
)


_CMB_SUB = 25
_CMB_BLK = _CMB_SUB * _CHUNK


def _combine_body(mt_ref, g_ref, wmt_ref, o_ref):
    mwt = jnp.dot(wmt_ref[...], mt_ref[...], preferred_element_type=jnp.float32)
    eye = jnp.eye(128, dtype=jnp.float32)
    gt = lax.dot_general(eye, g_ref[...], (((1,), (1,)), ((), ())),
                         preferred_element_type=jnp.float32)
    for kb in range(_CMB_SUB):
        for k in range(8):
            c = kb * _CHUNK + k * _P
            o_ref[:, c:c + _P] = (
                mwt[:, c:c + _P]
                + gt[16 * k:16 * (k + 1), kb * _P:(kb + 1) * _P])


def _make_combine(grid_off, grid_n, aliased):
    nrows = grid_n * _CMB_SUB * _P

    def body(*refs):
        if aliased:
            _combine_body(*refs[1:])
        else:
            _combine_body(*refs)

    in_specs = [
        pl.BlockSpec((16, _CMB_BLK), lambda i: (0, i + grid_off)),
        pl.BlockSpec((nrows // grid_n, 128), lambda i: (i, 0)),
        pl.BlockSpec((_DM, _DOUT), lambda i: (0, 0)),
    ]
    kwargs = {}
    if aliased:
        in_specs = [pl.BlockSpec(memory_space=pl.ANY)] + in_specs
        kwargs["input_output_aliases"] = {0: 0}
    return pl.pallas_call(
        body,
        grid=(grid_n,),
        in_specs=in_specs,
        out_specs=pl.BlockSpec((16, _CMB_BLK), lambda i: (0, i + grid_off)),
        out_shape=jax.ShapeDtypeStruct((16, _E), jnp.float32),
        **kwargs,
    )


_combine0 = _make_combine(0, _PART0_CHUNKS // _CMB_SUB, aliased=False)
_combine1 = _make_combine(_PART0_CHUNKS // _CMB_SUB, _PART1_CHUNKS // _CMB_SUB,
                          aliased=True)


def kernel(h, m_flux, v, edges, W, b):
    wh = W[:_D]
    wm = W[_D:_D + _DM]
    wvs = W[_D + _DM:_D + _DM + _D]
    wvd = W[_D + _DM + _D:]
    a_tab, b_tab = _node_proj(h, v, wh, wvs, wvd, b.reshape(1, _DOUT))
    edges_t = jnp.transpose(edges)
    g0 = _sc_part0(a_tab, b_tab, edges_t)
    g1 = _sc_part1(a_tab, b_tab, edges_t)
    mt = jnp.transpose(m_flux)
    wmt = jnp.transpose(wm)
    out_t = _combine0(mt, g0, wmt)
    out_t = _combine1(out_t, mt, g1, wmt)
    return jnp.transpose(out_t)


# speedup vs baseline: 1.0258x; 1.0258x over previous
import functools

import jax
import jax.numpy as jnp
from jax import lax
from jax.experimental import pallas as pl
from jax.experimental.pallas import tpu as pltpu
from jax.experimental.pallas import tpu_sc as plsc

_N, _E, _D, _DM, _DOUT = 10000, 640000, 128, 16, 16


def _node_proj_body(h_ref, v_ref, wh_ref, wvs_ref, wvd_ref, b_ref, a_ref, bt_ref):
    hW = jnp.dot(h_ref[...], wh_ref[...], preferred_element_type=jnp.float32)
    a_ref[...] = hW + jnp.dot(v_ref[...], wvs_ref[...],
                              preferred_element_type=jnp.float32)
    bt_ref[...] = (hW + jnp.dot(v_ref[...], wvd_ref[...],
                                preferred_element_type=jnp.float32)
                   + b_ref[...])


_NODE_BLK = 2000

_node_proj = pl.pallas_call(
    _node_proj_body,
    grid=(_N // _NODE_BLK,),
    in_specs=[
        pl.BlockSpec((_NODE_BLK, _D), lambda i: (i, 0)),
        pl.BlockSpec((_NODE_BLK, _D), lambda i: (i, 0)),
        pl.BlockSpec((_D, _DOUT), lambda i: (0, 0)),
        pl.BlockSpec((_D, _DOUT), lambda i: (0, 0)),
        pl.BlockSpec((_D, _DOUT), lambda i: (0, 0)),
        pl.BlockSpec((1, _DOUT), lambda i: (0, 0)),
    ],
    out_specs=[
        pl.BlockSpec((_NODE_BLK, _DOUT), lambda i: (i, 0)),
        pl.BlockSpec((_NODE_BLK, _DOUT), lambda i: (i, 0)),
    ],
    out_shape=[
        jax.ShapeDtypeStruct((_N, _DOUT), jnp.float32),
        jax.ShapeDtypeStruct((_N, _DOUT), jnp.float32),
    ],
)


_NW = 32
_CHUNK = 1024
_P = _CHUNK // 8
_NCHUNKS = _E // _CHUNK
_CPW = -(-_NCHUNKS // _NW)

_sc_mesh = plsc.VectorSubcoreMesh(core_axis_name="c", subcore_axis_name="s")


def _make_sc(chunk_off, nchunks):
    nfull = nchunks // _NW
    extra = nchunks - nfull * _NW

    @functools.partial(
        pl.kernel,
        out_type=jax.ShapeDtypeStruct((nchunks * _P, 128), jnp.float32),
        mesh=_sc_mesh,
        scratch_types=[
            pltpu.VMEM((_CHUNK,), jnp.int32),
            pltpu.VMEM((_CHUNK,), jnp.int32),
            pltpu.VMEM((_CHUNK,), jnp.int32),
            pltpu.VMEM((_CHUNK,), jnp.int32),
            pltpu.VMEM((_CHUNK, _DOUT), jnp.float32),
            pltpu.VMEM((_CHUNK, _DOUT), jnp.float32),
            pltpu.VMEM((_CHUNK, _DOUT), jnp.float32),
            pltpu.VMEM((_CHUNK, _DOUT), jnp.float32),
            pltpu.VMEM((_P, 128), jnp.float32),
            pltpu.VMEM((_P, 128), jnp.float32),
            pltpu.SemaphoreType.DMA,
            pltpu.SemaphoreType.DMA,
            pltpu.SemaphoreType.DMA,
            pltpu.SemaphoreType.DMA,
            pltpu.SemaphoreType.DMA,
            pltpu.SemaphoreType.DMA,
        ],
        compiler_params=pltpu.CompilerParams(use_tc_tiling_on_sc=False, needs_layout_passes=False),
    )
    def _sc_gather_sum(a_hbm, b_hbm, et_hbm, out_hbm,
                       idxs0, idxd0, idxs1, idxd1, bufa0, bufb0, bufa1, bufb1,
                       bufo0, bufo1, sem_i0, sem_i1, sem_g0, sem_g1, sem_o0, sem_o1):
        wid = lax.axis_index("s") * 2 + lax.axis_index("c")
        nw = jnp.where(wid < extra, nfull + 1, nfull)
        start = wid * nfull + jnp.minimum(wid, extra)
        idxs = (idxs0, idxs1)
        idxd = (idxd0, idxd1)
        bufa = (bufa0, bufa1)
        bufb = (bufb0, bufb1)
        bufo = (bufo0, bufo1)
        sem_i = (sem_i0, sem_i1)
        sem_g = (sem_g0, sem_g1)
        sem_o = (sem_o0, sem_o1)

        def issue_idx(p, ci):
            base = (chunk_off + start + ci) * _CHUNK
            pltpu.async_copy(et_hbm.at[0, pl.ds(base, _CHUNK)], idxs[p], sem_i[p])
            pltpu.async_copy(et_hbm.at[1, pl.ds(base, _CHUNK)], idxd[p], sem_i[p])

        def wait_idx(p):
            pltpu.make_async_copy(et_hbm.at[0, pl.ds(0, _CHUNK)], idxs[p], sem_i[p]).wait()
            pltpu.make_async_copy(et_hbm.at[1, pl.ds(0, _CHUNK)], idxd[p], sem_i[p]).wait()

        def issue_gather(p):
            pltpu.async_copy(a_hbm.at[idxs[p]], bufa[p], sem_g[p])
            pltpu.async_copy(b_hbm.at[idxd[p]], bufb[p], sem_g[p])

        def wait_gather(p):
            pltpu.make_async_copy(a_hbm.at[pl.ds(0, _CHUNK)], bufa[p], sem_g[p]).wait()
            pltpu.make_async_copy(b_hbm.at[pl.ds(0, _CHUNK)], bufb[p], sem_g[p]).wait()

        def issue_out(p, ci):
            pltpu.async_copy(bufo[p], out_hbm.at[pl.ds((start + ci) * _P, _P)], sem_o[p])

        def wait_out(p):
            pltpu.make_async_copy(bufo[p], out_hbm.at[pl.ds(0, _P)], sem_o[p]).wait()

        issue_idx(0, 0)
        wait_idx(0)
        issue_gather(0)
        issue_idx(1, 1)

        def pair_body(ci2, carry):
            for p in (0, 1):
                ci = ci2 * 2 + p

                @pl.when(ci < nw)
                def _():
                    op = 1 - p

                    @pl.when(ci + 1 < nw)
                    def _():
                        wait_idx(op)
                        issue_gather(op)

                    wait_gather(p)

                    @pl.when(ci + 2 < nw)
                    def _():
                        issue_idx(p, ci + 2)

                    def row_body(q, c2):
                        for k in range(8):
                            bufo[p][q, pl.ds(k * _DOUT, _DOUT)] = (
                                bufa[p][k * _P + q, :] + bufb[p][k * _P + q, :])
                        return c2

                    lax.fori_loop(0, _P, row_body, 0)

                    @pl.when(ci >= 2)
                    def _():
                        wait_out(p)

                    issue_out(p, ci)

            return carry

        lax.fori_loop(0, (nfull + 2) // 2, pair_body, 0)
        wait_out(0)
        wait_out(1)

    return _sc_gather_sum


_sc_all = _make_sc(0, _NCHUNKS)


_FMT_BLK = 12800


def _edge_fmt_body(ed_ref, t_ref):
    t_ref[...] = ed_ref[...].T


_edge_fmt = pl.pallas_call(
    _edge_fmt_body,
    grid=(_E // _FMT_BLK,),
    in_specs=[pl.BlockSpec((_FMT_BLK, 2), lambda i: (i, 0))],
    out_specs=pl.BlockSpec((2, _FMT_BLK), lambda i: (0, i)),
    out_shape=jax.ShapeDtypeStruct((2, _E), jnp.int32),
)


_CMB_SUB = 25
_CMB_BLK = _CMB_SUB * _CHUNK


def _combine_body(mt_ref, g_ref, wmt_ref, o_ref):
    mwt = jnp.dot(wmt_ref[...], mt_ref[...], preferred_element_type=jnp.float32)
    eye = jnp.eye(128, dtype=jnp.float32)
    gt = lax.dot_general(eye, g_ref[...], (((1,), (1,)), ((), ())),
                         preferred_element_type=jnp.float32)
    for kb in range(_CMB_SUB):
        for k in range(8):
            c = kb * _CHUNK + k * _P
            o_ref[:, c:c + _P] = (
                mwt[:, c:c + _P]
                + gt[16 * k:16 * (k + 1), kb * _P:(kb + 1) * _P])


def _make_combine(grid_off, grid_n, aliased):
    nrows = grid_n * _CMB_SUB * _P

    def body(*refs):
        if aliased:
            _combine_body(*refs[1:])
        else:
            _combine_body(*refs)

    in_specs = [
        pl.BlockSpec((16, _CMB_BLK), lambda i: (0, i + grid_off)),
        pl.BlockSpec((nrows // grid_n, 128), lambda i: (i, 0)),
        pl.BlockSpec((_DM, _DOUT), lambda i: (0, 0)),
    ]
    kwargs = {}
    if aliased:
        in_specs = [pl.BlockSpec(memory_space=pl.ANY)] + in_specs
        kwargs["input_output_aliases"] = {0: 0}
    return pl.pallas_call(
        body,
        grid=(grid_n,),
        in_specs=in_specs,
        out_specs=pl.BlockSpec((16, _CMB_BLK), lambda i: (0, i + grid_off)),
        out_shape=jax.ShapeDtypeStruct((16, _E), jnp.float32),
        **kwargs,
    )


_combine = _make_combine(0, _NCHUNKS // _CMB_SUB, aliased=False)


def kernel(h, m_flux, v, edges, W, b):
    wh = W[:_D]
    wm = W[_D:_D + _DM]
    wvs = W[_D + _DM:_D + _DM + _D]
    wvd = W[_D + _DM + _D:]
    a_tab, b_tab = _node_proj(h, v, wh, wvs, wvd, b.reshape(1, _DOUT))
    edges_t = jnp.transpose(edges)
    gsum = _sc_all(a_tab, b_tab, edges_t)
    out_t = _combine(jnp.transpose(m_flux), gsum, jnp.transpose(wm))
    return jnp.transpose(out_t)
